# Initial kernel scaffold; baseline (speedup 1.0000x reference)
import jax
import jax.numpy as jnp
from jax.experimental import pallas as pl


def kernel(node_embedding, task_embedding, edge_index, batch, W1, b1, W2, b2, P0w, P0b, P1w, P1b, P2w, P2b):
    raise NotImplementedError("write your pallas kernel here")


# speedup vs baseline: 7.4978x; 7.4978x over previous
import functools

import jax
import jax.numpy as jnp
from jax import lax
from jax.experimental import pallas as pl
from jax.experimental.pallas import tpu as pltpu
from jax.experimental.pallas import tpu_sc as plsc

N = 10000
E = 160000
D = 256
H = 256
G = 64

HALF = 128
NTILE = 16
CHUNK = 128
NCHUNK = 80
EPAD = NTILE * NCHUNK * CHUNK
NPAD = 10112
NROWB = 10
RB = N // NROWB


def _fill_const(ref, rows, value):
    cols = ref.shape[1] // 16

    def body(r, _):
        def inner(q, __):
            ref[r, pl.ds(q * 16, 16)] = jnp.full((16,), value, jnp.float32)
            return __
        return lax.fori_loop(0, cols, inner, 0)

    lax.fori_loop(0, rows, body, 0)


def _deg_body(dstr_hbm, deg_out, dbuf, ones_v, stage_v, deg_sh):
    c = lax.axis_index("c")
    s = lax.axis_index("s")

    _fill_const(ones_v, CHUNK, 1.0)
    _fill_const(stage_v, CHUNK, 0.0)

    def zloop(k, cr):
        j = s + NTILE * k

        @pl.when(j < NPAD // CHUNK)
        def _zero():
            pltpu.sync_copy(stage_v, deg_sh.at[pl.ds(j * CHUNK, CHUNK)])
        return cr
    lax.fori_loop(0, 5, zloop, 0)
    plsc.subcore_barrier()

    pltpu.sync_copy(dstr_hbm.at[s], dbuf)

    def eloop(k, cr):
        j = 2 * k + c
        pltpu.sync_copy(ones_v, deg_sh.at[dbuf.at[j]], add=True)
        return cr
    lax.fori_loop(0, NCHUNK // 2, eloop, 0)
    plsc.subcore_barrier()

    def xloop(k, cr):
        j = s + NTILE * k

        @pl.when(j < NPAD // CHUNK)
        def _exp():
            pltpu.sync_copy(deg_sh.at[pl.ds(j * CHUNK, CHUNK)], stage_v)
            pltpu.sync_copy(stage_v,
                            deg_out.at[c].at[pl.ds(j * CHUNK, CHUNK)])
        return cr
    lax.fori_loop(0, 5, xloop, 0)


def _make_deg_kernel():
    mesh = plsc.VectorSubcoreMesh(core_axis_name="c", subcore_axis_name="s")
    return pl.kernel(
        _deg_body,
        out_type=jax.ShapeDtypeStruct((2, NPAD, HALF), jnp.float32),
        mesh=mesh,
        scratch_types=[
            pltpu.VMEM((NCHUNK, CHUNK), jnp.int32),
            pltpu.VMEM((CHUNK, HALF), jnp.float32),
            pltpu.VMEM((CHUNK, HALF), jnp.float32),
            pltpu.VMEM_SHARED((NPAD, HALF), jnp.float32),
        ],
    )


def _edge_body(hs_a_hbm, hs_b_hbm, srcr_hbm, dstr_hbm, out_a, out_b,
               src_v, dst_v, buf, acc_sh, sem):
    c = lax.axis_index("c")
    s = lax.axis_index("s")

    _fill_const(buf, CHUNK, 0.0)

    def zloop(k, cr):
        j = s + NTILE * k

        @pl.when(j < NPAD // CHUNK)
        def _zero():
            pltpu.sync_copy(buf, acc_sh.at[pl.ds(j * CHUNK, CHUNK)])
        return cr
    lax.fori_loop(0, 5, zloop, 0)
    plsc.subcore_barrier()

    pltpu.sync_copy(srcr_hbm.at[s], src_v)
    pltpu.sync_copy(dstr_hbm.at[s], dst_v)

    def eloop(j, cr):
        @pl.when(c == 0)
        def _g0():
            pltpu.async_copy(hs_a_hbm.at[src_v.at[j]], buf, sem).wait()

        @pl.when(c == 1)
        def _g1():
            pltpu.async_copy(hs_b_hbm.at[src_v.at[j]], buf, sem).wait()

        pltpu.sync_copy(buf, acc_sh.at[dst_v.at[j]], add=True)
        return cr
    lax.fori_loop(0, NCHUNK, eloop, 0)
    plsc.subcore_barrier()

    def xloop(k, cr):
        j = s + NTILE * k

        @pl.when(j < 78)
        def _full():
            pltpu.sync_copy(acc_sh.at[pl.ds(j * CHUNK, CHUNK)], buf)

            @pl.when(c == 0)
            def _f0():
                pltpu.sync_copy(buf, out_a.at[pl.ds(j * CHUNK, CHUNK)])

            @pl.when(c == 1)
            def _f1():
                pltpu.sync_copy(buf, out_b.at[pl.ds(j * CHUNK, CHUNK)])

        @pl.when(j == 78)
        def _tail():
            pltpu.sync_copy(acc_sh.at[pl.ds(78 * CHUNK, 16)],
                            buf.at[pl.ds(0, 16)])

            @pl.when(c == 0)
            def _t0():
                pltpu.sync_copy(buf.at[pl.ds(0, 16)],
                                out_a.at[pl.ds(78 * CHUNK, 16)])

            @pl.when(c == 1)
            def _t1():
                pltpu.sync_copy(buf.at[pl.ds(0, 16)],
                                out_b.at[pl.ds(78 * CHUNK, 16)])
        return cr
    lax.fori_loop(0, 5, xloop, 0)


def _make_edge_kernel():
    mesh = plsc.VectorSubcoreMesh(core_axis_name="c", subcore_axis_name="s")
    return pl.kernel(
        _edge_body,
        out_type=[jax.ShapeDtypeStruct((N, HALF), jnp.float32),
                  jax.ShapeDtypeStruct((N, HALF), jnp.float32)],
        mesh=mesh,
        scratch_types=[
            pltpu.VMEM((NCHUNK, CHUNK), jnp.int32),
            pltpu.VMEM((NCHUNK, CHUNK), jnp.int32),
            pltpu.VMEM((CHUNK, HALF), jnp.float32),
            pltpu.VMEM_SHARED((NPAD, HALF), jnp.float32),
            pltpu.SemaphoreType.DMA,
        ],
    )


def _dinv_of(deg_ref):
    deg = deg_ref[0, :, 0:1] + deg_ref[1, :, 0:1] + 1.0
    return lax.rsqrt(jnp.maximum(deg, 1e-12))


def _tc1_body(x_ref, w1_ref, deg_ref, m_ref, ha_ref, hb_ref):
    dinv = _dinv_of(deg_ref)
    m = jnp.dot(x_ref[...], w1_ref[...], preferred_element_type=jnp.float32)
    m_ref[...] = m
    hs = m * dinv
    ha_ref[...] = hs[:, :HALF]
    hb_ref[...] = hs[:, HALF:]


def _tc2_body(m_ref, aa_ref, ab_ref, deg_ref, b1_ref, w2_ref,
              m2_ref, ha_ref, hb_ref):
    dinv = _dinv_of(deg_ref)
    acc = jnp.concatenate([aa_ref[...], ab_ref[...]], axis=1)
    out1 = dinv * acc + (dinv * dinv) * m_ref[...] + b1_ref[...]
    h = jnp.maximum(out1, 0.0)
    m2 = jnp.dot(h, w2_ref[...], preferred_element_type=jnp.float32)
    m2_ref[...] = m2
    hs = m2 * dinv
    ha_ref[...] = hs[:, :HALF]
    hb_ref[...] = hs[:, HALF:]


def _tc3_body(m_ref, aa_ref, ab_ref, deg_ref, b2_ref, batch_ref,
              task_ref, p0w_ref, p0b_ref, p1w_ref, p1b_ref, p2w_ref, p2b_ref,
              score_ref, sums_ref, cnt_ref):
    i = pl.program_id(0)
    dinv = _dinv_of(deg_ref)
    acc = jnp.concatenate([aa_ref[...], ab_ref[...]], axis=1)
    out2 = dinv * acc + (dinv * dinv) * m_ref[...] + b2_ref[...]

    iota = lax.broadcasted_iota(jnp.int32, (RB, G), 1)
    onehot = (batch_ref[...] == iota).astype(jnp.float32)
    contract = (((0,), (0,)), ((), ()))
    psums = lax.dot_general(onehot, out2, contract,
                            preferred_element_type=jnp.float32)
    pcnt = lax.dot_general(onehot, jnp.ones((RB, 128), jnp.float32), contract,
                           preferred_element_type=jnp.float32)

    @pl.when(i == 0)
    def _():
        sums_ref[...] = jnp.zeros_like(sums_ref)
        cnt_ref[...] = jnp.zeros_like(cnt_ref)

    sums_ref[...] += psums
    cnt_ref[...] += pcnt

    @pl.when(i == NROWB - 1)
    def _():
        cnt = cnt_ref[:, 0:1]
        pooled = sums_ref[...] / jnp.maximum(cnt, 1.0)
        nrm = jnp.sqrt(jnp.sum(pooled * pooled, axis=1, keepdims=True))
        pooled = pooled / jnp.maximum(nrm, 1e-12)

        t = jnp.dot(task_ref[...], p0w_ref[...],
                    preferred_element_type=jnp.float32) + p0b_ref[...]
        t = jnp.maximum(t, 0.0)
        t = jnp.dot(t, p1w_ref[...],
                    preferred_element_type=jnp.float32) + p1b_ref[...]
        t = jnp.maximum(t, 0.0)
        t = jnp.dot(t, p2w_ref[...],
                    preferred_element_type=jnp.float32) + p2b_ref[...]
        tn = jnp.sqrt(jnp.sum(t * t, axis=1, keepdims=True))
        t = t / jnp.maximum(tn, 1e-12)

        sc = jnp.sum(pooled * t, axis=1, keepdims=True)
        sig = 1.0 / (1.0 + jnp.exp(-sc))
        score_ref[...] = jnp.broadcast_to(sig, (G, 128))


def _row_spec(w):
    return pl.BlockSpec((RB, w), lambda i: (i, 0))


def _deg_spec():
    return pl.BlockSpec((2, RB, HALF), lambda i: (0, i, 0))


def _full_spec(r, w):
    return pl.BlockSpec((r, w), lambda i: (0, 0))


def _make_tc1():
    return pl.pallas_call(
        _tc1_body,
        grid=(NROWB,),
        in_specs=[_row_spec(D), _full_spec(D, H), _deg_spec()],
        out_specs=[_row_spec(H), _row_spec(HALF), _row_spec(HALF)],
        out_shape=[jax.ShapeDtypeStruct((N, H), jnp.float32),
                   jax.ShapeDtypeStruct((N, HALF), jnp.float32),
                   jax.ShapeDtypeStruct((N, HALF), jnp.float32)],
    )


def _make_tc2():
    return pl.pallas_call(
        _tc2_body,
        grid=(NROWB,),
        in_specs=[_row_spec(H), _row_spec(HALF), _row_spec(HALF),
                  _deg_spec(), _full_spec(1, H),
                  _full_spec(H, H)],
        out_specs=[_row_spec(H), _row_spec(HALF), _row_spec(HALF)],
        out_shape=[jax.ShapeDtypeStruct((N, H), jnp.float32),
                   jax.ShapeDtypeStruct((N, HALF), jnp.float32),
                   jax.ShapeDtypeStruct((N, HALF), jnp.float32)],
    )


def _make_tc3():
    return pl.pallas_call(
        _tc3_body,
        grid=(NROWB,),
        in_specs=[_row_spec(H), _row_spec(HALF), _row_spec(HALF),
                  _deg_spec(), _full_spec(1, H),
                  _row_spec(1), _full_spec(G, D),
                  _full_spec(D, H), _full_spec(1, H),
                  _full_spec(H, H), _full_spec(1, H),
                  _full_spec(H, H), _full_spec(1, H)],
        out_specs=[_full_spec(G, 128)],
        out_shape=[jax.ShapeDtypeStruct((G, 128), jnp.float32)],
        scratch_shapes=[pltpu.VMEM((G, H), jnp.float32),
                        pltpu.VMEM((G, 128), jnp.float32)],
    )


@jax.jit
def kernel(node_embedding, task_embedding, edge_index, batch,
           W1, b1, W2, b2, P0w, P0b, P1w, P1b, P2w, P2b):
    src = edge_index[0]
    dst = edge_index[1]
    srcp = jnp.pad(src, (0, EPAD - E)).reshape(NTILE, NCHUNK, CHUNK)
    dstp = jnp.pad(dst, (0, EPAD - E),
                   constant_values=N).reshape(NTILE, NCHUNK, CHUNK)

    deg_kernel = _make_deg_kernel()
    edge_kernel = _make_edge_kernel()
    tc1 = _make_tc1()
    tc2 = _make_tc2()
    tc3 = _make_tc3()

    deg = deg_kernel(dstp)
    m1, hs1a, hs1b = tc1(node_embedding, W1, deg)
    acc1a, acc1b = edge_kernel(hs1a, hs1b, srcp, dstp)
    m2, hs2a, hs2b = tc2(m1, acc1a, acc1b, deg,
                         b1.reshape(1, H), W2)
    acc2a, acc2b = edge_kernel(hs2a, hs2b, srcp, dstp)
    (score,) = tc3(m2, acc2a, acc2b, deg, b2.reshape(1, H),
                   batch.reshape(N, 1), task_embedding,
                   P0w, P0b.reshape(1, H), P1w, P1b.reshape(1, H),
                   P2w, P2b.reshape(1, H))
    return score[:, 0]


# speedup vs baseline: 8.0502x; 1.0737x over previous
import jax
import jax.numpy as jnp
from jax import lax
from jax.experimental import pallas as pl
from jax.experimental.pallas import tpu as pltpu
from jax.experimental.pallas import tpu_sc as plsc

N = 10000
E = 160000
D = 256
H = 256
G = 64

HALF = 128
NTILE = 16
CHUNK = 128
NCHUNK = 80
NBUF = 2
GRP = 16
EPAD = NTILE * NCHUNK * CHUNK
NPAD = 10112
NROWB = 10
RB = N // NROWB


def _fill_const(ref, rows, value):
    cols = ref.shape[1] // 16

    def body(r, cr):
        def inner(q, cr2):
            ref[r, pl.ds(q * 16, 16)] = jnp.full((16,), value, jnp.float32)
            return cr2
        return lax.fori_loop(0, cols, inner, cr)

    lax.fori_loop(0, rows, body, 0)


def _deg_body(dstr_hbm, deg_out, dbuf, ones_v, stage_v, deg_sh):
    c = lax.axis_index("c")
    s = lax.axis_index("s")

    _fill_const(ones_v, CHUNK, 1.0)
    _fill_const(stage_v, CHUNK, 0.0)

    def zloop(k, cr):
        j = s + NTILE * k

        @pl.when(j < NPAD // CHUNK)
        def _zero():
            pltpu.sync_copy(stage_v, deg_sh.at[pl.ds(j * CHUNK, CHUNK)])
        return cr
    lax.fori_loop(0, 5, zloop, 0)
    plsc.subcore_barrier()

    pltpu.sync_copy(dstr_hbm.at[s], dbuf)

    def eloop(k, cr):
        j = 2 * k + c
        pltpu.sync_copy(ones_v, deg_sh.at[dbuf.at[j]], add=True)
        return cr
    lax.fori_loop(0, NCHUNK // 2, eloop, 0)
    plsc.subcore_barrier()

    def xloop(k, cr):
        j = s + NTILE * k

        @pl.when(j < NPAD // CHUNK)
        def _exp():
            pltpu.sync_copy(deg_sh.at[pl.ds(j * CHUNK, CHUNK)], stage_v)
            pltpu.sync_copy(stage_v,
                            deg_out.at[c].at[pl.ds(j * CHUNK, CHUNK)])
        return cr
    lax.fori_loop(0, 5, xloop, 0)


def _make_deg_kernel():
    mesh = plsc.VectorSubcoreMesh(core_axis_name="c", subcore_axis_name="s")
    return pl.kernel(
        _deg_body,
        out_type=jax.ShapeDtypeStruct((2, NPAD, HALF), jnp.float32),
        mesh=mesh,
        scratch_types=[
            pltpu.VMEM((NCHUNK, CHUNK), jnp.int32),
            pltpu.VMEM((CHUNK, HALF), jnp.float32),
            pltpu.VMEM((CHUNK, HALF), jnp.float32),
            pltpu.VMEM_SHARED((NPAD, HALF), jnp.float32),
        ],
    )


def _edge_body(tab_hbm, srcr_hbm, dstr_hbm, out,
               src_v, dst_v, b0, b1, acc_sh,
               g0, g1, s0, s1):
    c = lax.axis_index("c")
    s = lax.axis_index("s")
    bufs = (b0, b1)
    gsem = (g0, g1)
    ssem = (s0, s1)

    _fill_const(b0, CHUNK, 0.0)

    def zloop(k, cr):
        j = s + NTILE * k

        @pl.when(j < NPAD // CHUNK)
        def _zero():
            pltpu.sync_copy(b0, acc_sh.at[pl.ds(j * CHUNK, CHUNK)])
        return cr
    lax.fori_loop(0, 5, zloop, 0)

    plsc.subcore_barrier()

    def gloop(g, cr):
        pltpu.sync_copy(srcr_hbm.at[s].at[pl.ds(g * GRP, GRP)], src_v)
        pltpu.sync_copy(dstr_hbm.at[s].at[pl.ds(g * GRP, GRP)], dst_v)

        def adj(r, cr2):
            def adj2(q, cr3):
                v = src_v[r, pl.ds(q * 16, 16)]
                src_v[r, pl.ds(q * 16, 16)] = v + c * N
                return cr3
            return lax.fori_loop(0, CHUNK // 16, adj2, cr2)
        lax.fori_loop(0, GRP, adj, 0)

        for i in range(NBUF):
            pltpu.async_copy(tab_hbm.at[src_v.at[i]], bufs[i], gsem[i])

        def eloop(k, cr2):
            for i in range(NBUF):
                j = NBUF * k + i
                pltpu.make_async_copy(
                    tab_hbm.at[src_v.at[j]], bufs[i], gsem[i]).wait()
                pltpu.async_copy(bufs[i], acc_sh.at[dst_v.at[j]], ssem[i],
                                 add=True)
            for i in range(NBUF):
                j = NBUF * k + i
                pltpu.make_async_copy(
                    bufs[i], acc_sh.at[dst_v.at[j]], ssem[i]).wait()
                jn = j + NBUF

                @pl.when(jn < GRP)
                def _nx(i=i, jn=jn):
                    pltpu.async_copy(tab_hbm.at[src_v.at[jn]], bufs[i],
                                     gsem[i])
            return cr2
        lax.fori_loop(0, GRP // NBUF, eloop, 0)
        return cr
    lax.fori_loop(0, NCHUNK // GRP, gloop, 0)
    plsc.subcore_barrier()

    def xloop(k, cr):
        j = s + NTILE * k

        @pl.when(j < 78)
        def _full():
            pltpu.sync_copy(acc_sh.at[pl.ds(j * CHUNK, CHUNK)], b0)
            pltpu.sync_copy(b0, out.at[c].at[pl.ds(j * CHUNK, CHUNK)])

        @pl.when(j == 78)
        def _tail():
            pltpu.sync_copy(acc_sh.at[pl.ds(78 * CHUNK, 16)],
                            b0.at[pl.ds(0, 16)])
            pltpu.sync_copy(b0.at[pl.ds(0, 16)],
                            out.at[c].at[pl.ds(78 * CHUNK, 16)])
        return cr
    lax.fori_loop(0, 5, xloop, 0)


def _make_edge_kernel():
    mesh = plsc.VectorSubcoreMesh(core_axis_name="c", subcore_axis_name="s")
    return pl.kernel(
        _edge_body,
        out_type=jax.ShapeDtypeStruct((2, N, HALF), jnp.float32),
        mesh=mesh,
        scratch_types=[
            pltpu.VMEM((GRP, CHUNK), jnp.int32),
            pltpu.VMEM((GRP, CHUNK), jnp.int32),
            pltpu.VMEM((CHUNK, HALF), jnp.float32),
            pltpu.VMEM((CHUNK, HALF), jnp.float32),
            pltpu.VMEM_SHARED((NPAD, HALF), jnp.float32),
            pltpu.SemaphoreType.DMA, pltpu.SemaphoreType.DMA,
            pltpu.SemaphoreType.DMA, pltpu.SemaphoreType.DMA,
        ],
    )


def _dinv_of(deg_ref):
    deg = deg_ref[0, :, 0:1] + deg_ref[1, :, 0:1] + 1.0
    return lax.rsqrt(jnp.maximum(deg, 1e-12))


def _tc1_body(x_ref, w1_ref, deg_ref, m_ref, hs_ref):
    dinv = _dinv_of(deg_ref)
    m = jnp.dot(x_ref[...], w1_ref[...], preferred_element_type=jnp.float32)
    m_ref[...] = m
    hs = m * dinv
    hs_ref[0] = hs[:, :HALF]
    hs_ref[1] = hs[:, HALF:]


def _tc2_body(m_ref, acc_ref, deg_ref, b1_ref, w2_ref, m2_ref, hs_ref):
    dinv = _dinv_of(deg_ref)
    acc = jnp.concatenate([acc_ref[0], acc_ref[1]], axis=1)
    out1 = dinv * acc + (dinv * dinv) * m_ref[...] + b1_ref[...]
    h = jnp.maximum(out1, 0.0)
    m2 = jnp.dot(h, w2_ref[...], preferred_element_type=jnp.float32)
    m2_ref[...] = m2
    hs = m2 * dinv
    hs_ref[0] = hs[:, :HALF]
    hs_ref[1] = hs[:, HALF:]


def _tc3_body(m_ref, acc_ref, deg_ref, b2_ref, batch_ref,
              task_ref, p0w_ref, p0b_ref, p1w_ref, p1b_ref, p2w_ref, p2b_ref,
              score_ref, sums_ref, cnt_ref):
    i = pl.program_id(0)
    dinv = _dinv_of(deg_ref)
    acc = jnp.concatenate([acc_ref[0], acc_ref[1]], axis=1)
    out2 = dinv * acc + (dinv * dinv) * m_ref[...] + b2_ref[...]

    iota = lax.broadcasted_iota(jnp.int32, (RB, G), 1)
    onehot = (batch_ref[...] == iota).astype(jnp.float32)
    contract = (((0,), (0,)), ((), ()))
    psums = lax.dot_general(onehot, out2, contract,
                            preferred_element_type=jnp.float32)
    pcnt = lax.dot_general(onehot, jnp.ones((RB, 128), jnp.float32), contract,
                           preferred_element_type=jnp.float32)

    @pl.when(i == 0)
    def _init():
        sums_ref[...] = jnp.zeros_like(sums_ref)
        cnt_ref[...] = jnp.zeros_like(cnt_ref)

    sums_ref[...] += psums
    cnt_ref[...] += pcnt

    @pl.when(i == NROWB - 1)
    def _final():
        cnt = cnt_ref[:, 0:1]
        pooled = sums_ref[...] / jnp.maximum(cnt, 1.0)
        nrm = jnp.sqrt(jnp.sum(pooled * pooled, axis=1, keepdims=True))
        pooled = pooled / jnp.maximum(nrm, 1e-12)

        t = jnp.dot(task_ref[...], p0w_ref[...],
                    preferred_element_type=jnp.float32) + p0b_ref[...]
        t = jnp.maximum(t, 0.0)
        t = jnp.dot(t, p1w_ref[...],
                    preferred_element_type=jnp.float32) + p1b_ref[...]
        t = jnp.maximum(t, 0.0)
        t = jnp.dot(t, p2w_ref[...],
                    preferred_element_type=jnp.float32) + p2b_ref[...]
        tn = jnp.sqrt(jnp.sum(t * t, axis=1, keepdims=True))
        t = t / jnp.maximum(tn, 1e-12)

        sc = jnp.sum(pooled * t, axis=1, keepdims=True)
        sig = 1.0 / (1.0 + jnp.exp(-sc))
        score_ref[...] = jnp.broadcast_to(sig, (G, 128))


def _row_spec(w):
    return pl.BlockSpec((RB, w), lambda i: (i, 0))


def _stk_spec():
    return pl.BlockSpec((2, RB, HALF), lambda i: (0, i, 0))


def _full_spec(r, w):
    return pl.BlockSpec((r, w), lambda i: (0, 0))


def _make_tc1():
    return pl.pallas_call(
        _tc1_body,
        grid=(NROWB,),
        in_specs=[_row_spec(D), _full_spec(D, H), _stk_spec()],
        out_specs=[_row_spec(H), _stk_spec()],
        out_shape=[jax.ShapeDtypeStruct((N, H), jnp.float32),
                   jax.ShapeDtypeStruct((2, N, HALF), jnp.float32)],
    )


def _make_tc2():
    return pl.pallas_call(
        _tc2_body,
        grid=(NROWB,),
        in_specs=[_row_spec(H), _stk_spec(), _stk_spec(), _full_spec(1, H),
                  _full_spec(H, H)],
        out_specs=[_row_spec(H), _stk_spec()],
        out_shape=[jax.ShapeDtypeStruct((N, H), jnp.float32),
                   jax.ShapeDtypeStruct((2, N, HALF), jnp.float32)],
    )


def _make_tc3():
    return pl.pallas_call(
        _tc3_body,
        grid=(NROWB,),
        in_specs=[_row_spec(H), _stk_spec(), _stk_spec(), _full_spec(1, H),
                  _row_spec(1), _full_spec(G, D),
                  _full_spec(D, H), _full_spec(1, H),
                  _full_spec(H, H), _full_spec(1, H),
                  _full_spec(H, H), _full_spec(1, H)],
        out_specs=[_full_spec(G, 128)],
        out_shape=[jax.ShapeDtypeStruct((G, 128), jnp.float32)],
        scratch_shapes=[pltpu.VMEM((G, H), jnp.float32),
                        pltpu.VMEM((G, 128), jnp.float32)],
    )


@jax.jit
def kernel(node_embedding, task_embedding, edge_index, batch,
           W1, b1, W2, b2, P0w, P0b, P1w, P1b, P2w, P2b):
    src = edge_index[0]
    dst = edge_index[1]
    srcp = jnp.pad(src, (0, EPAD - E)).reshape(NTILE, NCHUNK, CHUNK)
    dstp = jnp.pad(dst, (0, EPAD - E),
                   constant_values=N).reshape(NTILE, NCHUNK, CHUNK)

    deg_kernel = _make_deg_kernel()
    edge_kernel = _make_edge_kernel()
    tc1 = _make_tc1()
    tc2 = _make_tc2()
    tc3 = _make_tc3()

    deg = deg_kernel(dstp)
    m1, hs1 = tc1(node_embedding, W1, deg)
    acc1 = edge_kernel(hs1.reshape(2 * N, HALF), srcp, dstp)
    m2, hs2 = tc2(m1, acc1, deg, b1.reshape(1, H), W2)
    acc2 = edge_kernel(hs2.reshape(2 * N, HALF), srcp, dstp)
    (score,) = tc3(m2, acc2, deg, b2.reshape(1, H),
                   batch.reshape(N, 1), task_embedding,
                   P0w, P0b.reshape(1, H), P1w, P1b.reshape(1, H),
                   P2w, P2b.reshape(1, H))
    return score[:, 0]
